# Initial kernel scaffold; baseline (speedup 1.0000x reference)
import jax
import jax.numpy as jnp
from jax.experimental import pallas as pl


def kernel(state, edge_index, deterministic, Wg, bg, W1, b1, W2, b2, W3, b3):
    raise NotImplementedError("write your pallas kernel here")


# speedup vs baseline: 14.3558x; 14.3558x over previous
import functools

import jax
import jax.numpy as jnp
from jax import lax
from jax.experimental import pallas as pl
from jax.experimental.pallas import tpu as pltpu
from jax.experimental.pallas import tpu_sc as plsc

N = 10000
E = 320000
D = 128
H = 32
ACT = 8

NC = 2
NS = 16
NW = NC * NS

CHUNK = 128
EPW = -(-E // NW)
C = -(-EPW // CHUNK)
E_PAD = NW * C * CHUNK
N_PAD = 10240
R = N_PAD // NS

_mesh = plsc.VectorSubcoreMesh(core_axis_name="c", subcore_axis_name="s")
L = 16
NR = 128
VPW = E_PAD // (NW * L)


NH = NR * D
BS = NH // NS


@functools.partial(
    pl.kernel,
    out_type=jax.ShapeDtypeStruct((NC, NH), jnp.float32),
    mesh=_mesh,
    scratch_types=[
        pltpu.VMEM((VPW, L), jnp.int32),
        pltpu.VMEM((NH,), jnp.float32),
        pltpu.VMEM((BS,), jnp.float32),
        pltpu.VMEM((BS,), jnp.float32),
        pltpu.VMEM_SHARED((NS, NH), jnp.float32),
    ],
    compiler_params=pltpu.CompilerParams(needs_layout_passes=False),
)
def _deg_sc(dst_hbm, zslots_hbm, deg_out, dst_v, hist_v, acc_v, tmp_v, deg_sh):
    c = lax.axis_index("c")
    s = lax.axis_index("s")
    wid = s * NC + c
    pltpu.sync_copy(zslots_hbm, hist_v)
    pltpu.sync_copy(dst_hbm.at[wid], dst_v)
    ones = jnp.ones((L,), jnp.float32)

    def body(i, carry):
        plsc.addupdate_scatter(hist_v, [dst_v[i]], ones)
        return carry

    lax.fori_loop(0, VPW, body, 0)
    pltpu.sync_copy(hist_v, deg_sh.at[s])
    plsc.subcore_barrier()
    pltpu.sync_copy(deg_sh.at[0, pl.ds(s * BS, BS)], acc_v)

    def red(t, carry):
        pltpu.sync_copy(deg_sh.at[t, pl.ds(s * BS, BS)], tmp_v)

        def add16(k, carry2):
            acc_v[pl.ds(k * L, L)] = acc_v[pl.ds(k * L, L)] + tmp_v[pl.ds(k * L, L)]
            return carry2

        lax.fori_loop(0, BS // L, add16, 0)
        return carry

    lax.fori_loop(1, NS, red, 0)
    pltpu.sync_copy(acc_v, deg_out.at[c, pl.ds(s * BS, BS)])


@functools.partial(
    pl.kernel,
    out_type=jax.ShapeDtypeStruct((NC, N_PAD, D), jnp.float32),
    mesh=_mesh,
    scratch_types=[
        pltpu.VMEM((C, CHUNK), jnp.int32),
        pltpu.VMEM((C, CHUNK), jnp.int32),
        pltpu.VMEM((CHUNK, D), jnp.float32),
        pltpu.VMEM_SHARED((N_PAD, D), jnp.float32),
    ],
)
def _edge_sc(y_hbm, src_hbm, dst_hbm, acc_out, src_v, dst_v, rows_v, acc_sh):
    c = lax.axis_index("c")
    s = lax.axis_index("s")
    wid = s * NC + c
    rs = s * R
    pltpu.sync_copy(y_hbm.at[pl.ds(rs, R)], acc_sh.at[pl.ds(rs, R)])
    pltpu.sync_copy(src_hbm.at[wid], src_v)
    pltpu.sync_copy(dst_hbm.at[wid], dst_v)
    plsc.subcore_barrier()

    def body(j, carry):
        pltpu.sync_copy(y_hbm.at[src_v.at[j]], rows_v)
        pltpu.sync_copy(rows_v, acc_sh.at[dst_v.at[j]], add=True)
        return carry

    lax.fori_loop(0, C, body, 0)
    plsc.subcore_barrier()
    pltpu.sync_copy(acc_sh.at[pl.ds(rs, R)], acc_out.at[c].at[pl.ds(rs, R)])


def _scale_body(state_ref, wg_ref, degp_ref, y_ref, dinv_ref):
    deg = degp_ref[0] + degp_ref[1] + 1.0
    dinv = lax.rsqrt(deg)
    xw = jnp.dot(state_ref[...], wg_ref[...], preferred_element_type=jnp.float32)
    y_ref[...] = xw * dinv
    dinv_ref[...] = dinv


def _scale_tc(state_p, Wg, degp):
    return pl.pallas_call(
        _scale_body,
        out_shape=(
            jax.ShapeDtypeStruct((N_PAD, D), jnp.float32),
            jax.ShapeDtypeStruct((N_PAD, 1), jnp.float32),
        ),
    )(state_p, Wg, degp)


def _leaky(x):
    return jnp.where(x > 0, x, 0.01 * x)


def _final_body(acc_ref, y_ref, state_ref, dinv_ref, bg_ref, w1_ref, b1_ref,
                w2_ref, b2_ref, w3_ref, b3_ref, out_ref):
    a = acc_ref[0, 0:N, :] + acc_ref[1, 0:N, :] - y_ref[0:N, :]
    g = a * dinv_ref[0:N, :] + bg_ref[...]
    g = jnp.maximum(g, 0.0) + state_ref[0:N, :]
    h = _leaky(jnp.dot(g, w1_ref[...], preferred_element_type=jnp.float32)
               + b1_ref[...])
    h = _leaky(jnp.dot(h, w2_ref[...], preferred_element_type=jnp.float32)
               + b2_ref[...])
    z = jnp.dot(h, w3_ref[...], preferred_element_type=jnp.float32) + b3_ref[...]
    conc = jnp.maximum(z, 0.0) + jnp.log1p(jnp.exp(-jnp.abs(z)))
    out_ref[...] = conc / (jnp.sum(conc) + 1e-20)


def _final_tc(accp, y, state_p, dinv, bg2, W1, b12, W2, b22, W3, b32):
    return pl.pallas_call(
        _final_body,
        out_shape=jax.ShapeDtypeStruct((N, 1), jnp.float32),
    )(accp, y, state_p, dinv, bg2, W1, b12, W2, b22, W3, b32)


def kernel(state, edge_index, deterministic, Wg, bg, W1, b1, W2, b2, W3, b3):
    del deterministic
    src = edge_index[0]
    dst = edge_index[1]
    pad = E_PAD - E
    src_p = jnp.concatenate(
        [src, jnp.zeros((pad,), jnp.int32)]).reshape(NW, C, CHUNK)
    dst_p = jnp.concatenate(
        [dst, jnp.full((pad,), N, jnp.int32)]).reshape(NW, C, CHUNK)
    state_p = jnp.pad(state, ((0, N_PAD - N), (0, 0)))

    zslots = jnp.zeros((NH,), jnp.float32)

    degp = _deg_sc(dst_p.reshape(NW, VPW, L), zslots)
    degp = degp.reshape(NC, NH, 1)[:, :N_PAD]
    y, dinv = _scale_tc(state_p, Wg, degp)
    accp = _edge_sc(y, src_p, dst_p)
    action = _final_tc(accp, y, state_p, dinv, bg.reshape(1, D),
                       W1, b1.reshape(1, H), W2, b2.reshape(1, H),
                       W3, b3.reshape(1, 1))
    return action.reshape(N // ACT, ACT)
